# baseline (speedup 1.0000x reference)
import numpy as np
import jax
import jax.numpy as jnp
from jax.experimental import pallas as pl


def _interp_weights(n: int, nn: int) -> np.ndarray:
    x_fix = np.arange(n, dtype=np.float64)
    x_var = np.linspace(0.0, float(n - 1), nn)
    x_repeat = np.tile(x_var[:, None], (len(x_fix),))
    distances = np.abs(x_repeat - x_fix)
    x_indices = np.searchsorted(x_fix, x_var)
    weights = np.zeros_like(distances)
    idx = np.arange(len(x_indices))
    weights[idx, x_indices] = distances[idx, x_indices - 1]
    weights[idx, x_indices - 1] = distances[idx, x_indices]
    weights /= np.sum(weights, axis=1)[:, None]
    return weights.astype(np.float32)


_N = 64
_NN = 128
_W_NP = _interp_weights(_N, _NN)


def _upsample_kernel(x_ref, w_ref, o_ref):
    n, nn = _N, _NN
    X = x_ref[0]
    W = w_ref[...]
    Wt = W.T
    hi = jax.lax.Precision.HIGHEST
    t = jax.lax.dot(X.reshape(n * n, n), Wt, precision=hi)
    t = t.reshape(n, n, nn)
    t = jnp.swapaxes(t, 1, 2)
    t = jax.lax.dot(t.reshape(n * nn, n), Wt, precision=hi)
    t = t.reshape(n, nn * nn)
    t = jax.lax.dot(W, t, precision=hi)
    o_ref[0] = jnp.swapaxes(t.reshape(nn, nn, nn), 1, 2)


def kernel(x):
    B, C, D, H, Wd = x.shape
    xs = x.reshape(C, D, H, Wd)
    w = jnp.asarray(_W_NP)
    out = pl.pallas_call(
        _upsample_kernel,
        grid=(C,),
        in_specs=[
            pl.BlockSpec((1, D, H, Wd), lambda c: (c, 0, 0, 0)),
            pl.BlockSpec((_NN, _N), lambda c: (0, 0)),
        ],
        out_specs=pl.BlockSpec((1, _NN, _NN, _NN), lambda c: (c, 0, 0, 0)),
        out_shape=jax.ShapeDtypeStruct((C, _NN, _NN, _NN), jnp.float32),
    )(xs, w)
    return out.reshape(B, C, _NN, _NN, _NN)


# speedup vs baseline: 1.0805x; 1.0805x over previous
import numpy as np
import jax
import jax.numpy as jnp
from jax.experimental import pallas as pl


def _interp_weights(n: int, nn: int) -> np.ndarray:
    x_fix = np.arange(n, dtype=np.float64)
    x_var = np.linspace(0.0, float(n - 1), nn)
    x_repeat = np.tile(x_var[:, None], (len(x_fix),))
    distances = np.abs(x_repeat - x_fix)
    x_indices = np.searchsorted(x_fix, x_var)
    weights = np.zeros_like(distances)
    idx = np.arange(len(x_indices))
    weights[idx, x_indices] = distances[idx, x_indices - 1]
    weights[idx, x_indices - 1] = distances[idx, x_indices]
    weights /= np.sum(weights, axis=1)[:, None]
    return weights.astype(np.float32)


_N = 64
_NN = 128
_W_NP = _interp_weights(_N, _NN)


def _upsample_kernel(x_ref, w_ref, o_ref):
    n, nn = _N, _NN
    X = x_ref[0]
    W = w_ref[...]
    Wt = W.T
    hi = jax.lax.Precision.HIGHEST
    t = jax.lax.dot(W, X.reshape(n, n * n), precision=hi)
    t = t.reshape(nn, n, n)
    t = jnp.swapaxes(t, 1, 2)
    t = jax.lax.dot(t.reshape(nn * n, n), Wt, precision=hi)
    t = jnp.swapaxes(t.reshape(nn, n, nn), 1, 2)
    o_ref[0] = jax.lax.dot(
        t.reshape(nn * nn, n), Wt, precision=hi).reshape(nn, nn, nn)


def kernel(x):
    B, C, D, H, Wd = x.shape
    xs = x.reshape(C, D, H, Wd)
    w = jnp.asarray(_W_NP)
    out = pl.pallas_call(
        _upsample_kernel,
        grid=(C,),
        in_specs=[
            pl.BlockSpec((1, D, H, Wd), lambda c: (c, 0, 0, 0)),
            pl.BlockSpec((_NN, _N), lambda c: (0, 0)),
        ],
        out_specs=pl.BlockSpec((1, _NN, _NN, _NN), lambda c: (c, 0, 0, 0)),
        out_shape=jax.ShapeDtypeStruct((C, _NN, _NN, _NN), jnp.float32),
    )(xs, w)
    return out.reshape(B, C, _NN, _NN, _NN)


# speedup vs baseline: 4.0251x; 3.7254x over previous
import numpy as np
import jax
import jax.numpy as jnp
from jax.experimental import pallas as pl


def _interp_weights(n: int, nn: int) -> np.ndarray:
    x_fix = np.arange(n, dtype=np.float64)
    x_var = np.linspace(0.0, float(n - 1), nn)
    x_repeat = np.tile(x_var[:, None], (len(x_fix),))
    distances = np.abs(x_repeat - x_fix)
    x_indices = np.searchsorted(x_fix, x_var)
    weights = np.zeros_like(distances)
    idx = np.arange(len(x_indices))
    weights[idx, x_indices] = distances[idx, x_indices - 1]
    weights[idx, x_indices - 1] = distances[idx, x_indices]
    weights /= np.sum(weights, axis=1)[:, None]
    return weights.astype(np.float32)


_N = 64
_NN = 128
_W_NP = _interp_weights(_N, _NN)


def _upsample_kernel(x_ref, w_ref, o_ref):
    n, nn = _N, _NN
    X = x_ref[0]
    W = w_ref[...]
    Wt = W.T

    def dot(a, b):
        return jax.lax.dot(a, b, preferred_element_type=jnp.float32)

    t = dot(W, X.reshape(n, n * n))
    t = t.reshape(nn, n, n)
    t = jnp.swapaxes(t, 1, 2)
    t = dot(t.reshape(nn * n, n), Wt)
    t = jnp.swapaxes(t.reshape(nn, n, nn), 1, 2)
    o_ref[0] = dot(t.reshape(nn * nn, n), Wt).reshape(nn, nn, nn)


def kernel(x):
    B, C, D, H, Wd = x.shape
    xs = x.reshape(C, D, H, Wd)
    w = jnp.asarray(_W_NP)
    out = pl.pallas_call(
        _upsample_kernel,
        grid=(C,),
        in_specs=[
            pl.BlockSpec((1, D, H, Wd), lambda c: (c, 0, 0, 0)),
            pl.BlockSpec((_NN, _N), lambda c: (0, 0)),
        ],
        out_specs=pl.BlockSpec((1, _NN, _NN, _NN), lambda c: (c, 0, 0, 0)),
        out_shape=jax.ShapeDtypeStruct((C, _NN, _NN, _NN), jnp.float32),
    )(xs, w)
    return out.reshape(B, C, _NN, _NN, _NN)


# speedup vs baseline: 5.2959x; 1.3157x over previous
import numpy as np
import jax
import jax.numpy as jnp
from jax.experimental import pallas as pl


def _interp_weights(n: int, nn: int) -> np.ndarray:
    x_fix = np.arange(n, dtype=np.float64)
    x_var = np.linspace(0.0, float(n - 1), nn)
    x_repeat = np.tile(x_var[:, None], (len(x_fix),))
    distances = np.abs(x_repeat - x_fix)
    x_indices = np.searchsorted(x_fix, x_var)
    weights = np.zeros_like(distances)
    idx = np.arange(len(x_indices))
    weights[idx, x_indices] = distances[idx, x_indices - 1]
    weights[idx, x_indices - 1] = distances[idx, x_indices]
    weights /= np.sum(weights, axis=1)[:, None]
    return weights.astype(np.float32)


_N = 64
_NN = 128
_W_NP = _interp_weights(_N, _NN)


def _upsample_kernel(x_ref, w_ref, o_ref):
    n, nn = _N, _NN
    X = x_ref[0].astype(jnp.bfloat16)
    W = w_ref[...].astype(jnp.bfloat16)
    Wt = W.T

    def dot(a, b):
        return jax.lax.dot(a, b, preferred_element_type=jnp.float32)

    t = dot(W, X.reshape(n, n * n)).astype(jnp.bfloat16)
    t = t.reshape(nn, n, n)
    t = jnp.swapaxes(t, 1, 2)
    t = dot(t.reshape(nn * n, n), Wt).astype(jnp.bfloat16)
    t = jnp.swapaxes(t.reshape(nn, n, nn), 1, 2)
    o_ref[0] = dot(t.reshape(nn * nn, n), Wt).reshape(nn, nn, nn)


def kernel(x):
    B, C, D, H, Wd = x.shape
    xs = x.reshape(C, D, H, Wd)
    w = jnp.asarray(_W_NP)
    out = pl.pallas_call(
        _upsample_kernel,
        grid=(C,),
        in_specs=[
            pl.BlockSpec((1, D, H, Wd), lambda c: (c, 0, 0, 0)),
            pl.BlockSpec((_NN, _N), lambda c: (0, 0)),
        ],
        out_specs=pl.BlockSpec((1, _NN, _NN, _NN), lambda c: (c, 0, 0, 0)),
        out_shape=jax.ShapeDtypeStruct((C, _NN, _NN, _NN), jnp.float32),
    )(xs, w)
    return out.reshape(B, C, _NN, _NN, _NN)


# speedup vs baseline: 5.5786x; 1.0534x over previous
import numpy as np
import jax
import jax.numpy as jnp
from jax.experimental import pallas as pl


def _interp_weights(n: int, nn: int) -> np.ndarray:
    x_fix = np.arange(n, dtype=np.float64)
    x_var = np.linspace(0.0, float(n - 1), nn)
    x_repeat = np.tile(x_var[:, None], (len(x_fix),))
    distances = np.abs(x_repeat - x_fix)
    x_indices = np.searchsorted(x_fix, x_var)
    weights = np.zeros_like(distances)
    idx = np.arange(len(x_indices))
    weights[idx, x_indices] = distances[idx, x_indices - 1]
    weights[idx, x_indices - 1] = distances[idx, x_indices]
    weights /= np.sum(weights, axis=1)[:, None]
    return weights.astype(np.float32)


_N = 64
_NN = 128
_W_NP = _interp_weights(_N, _NN)


def _upsample_kernel(x_ref, w_ref, o_ref):
    n, nn = _N, _NN
    X = x_ref[0].astype(jnp.bfloat16)
    W = w_ref[...].astype(jnp.bfloat16)
    Wt = W.T

    def dot(a, b):
        return jax.lax.dot(a, b, preferred_element_type=jnp.float32)

    t0 = dot(W, X.reshape(n, n * n)).astype(jnp.bfloat16)
    t0 = t0.reshape(nn, n, n)
    nc = 4
    pc = nn // nc
    for i in range(nc):
        t = t0[i * pc:(i + 1) * pc]
        t = jnp.swapaxes(t, 1, 2)
        t = dot(t.reshape(pc * n, n), Wt).astype(jnp.bfloat16)
        t = jnp.swapaxes(t.reshape(pc, n, nn), 1, 2)
        o_ref[0, i * pc:(i + 1) * pc] = dot(
            t.reshape(pc * nn, n), Wt).reshape(pc, nn, nn)


def kernel(x):
    B, C, D, H, Wd = x.shape
    xs = x.reshape(C, D, H, Wd)
    w = jnp.asarray(_W_NP)
    out = pl.pallas_call(
        _upsample_kernel,
        grid=(C,),
        in_specs=[
            pl.BlockSpec((1, D, H, Wd), lambda c: (c, 0, 0, 0)),
            pl.BlockSpec((_NN, _N), lambda c: (0, 0)),
        ],
        out_specs=pl.BlockSpec((1, _NN, _NN, _NN), lambda c: (c, 0, 0, 0)),
        out_shape=jax.ShapeDtypeStruct((C, _NN, _NN, _NN), jnp.float32),
    )(xs, w)
    return out.reshape(B, C, _NN, _NN, _NN)
